# baseline (speedup 1.0000x reference)
import functools

import jax
import jax.numpy as jnp
from jax import lax
from jax.experimental import pallas as pl
from jax.experimental.pallas import tpu as pltpu
from jax.experimental.pallas import tpu_sc as plsc

V = 100000
P = 8192
H = 1024
B = 4
S = 2048

NC = 2
NS = 16
NW = NC * NS
NT = B * S
TPW = NT // NW
CH = 16
NCHUNK = TPW // CH
LANES = 16
HCHUNKS = H // LANES


def _rsqrt(x):
    i = lax.bitcast_convert_type(x, jnp.int32)
    i = jnp.int32(0x5F3759DF) - lax.shift_right_arithmetic(i, jnp.int32(1))
    y = lax.bitcast_convert_type(i, jnp.float32)
    for _ in range(3):
        y = y * (jnp.float32(1.5) - jnp.float32(0.5) * x * y * y)
    return y


def _body(ids_hbm, word_hbm, pos_hbm, tt_hbm, gamma_hbm, beta_hbm, out_hbm,
          idx_v, rows_v, pos_v, tt_v, gamma_v, beta_v, sem):
    cid = lax.axis_index("c")
    sid = lax.axis_index("s")
    wid = sid * NC + cid
    base = pl.multiple_of(wid * TPW, TPW)
    s0 = pl.multiple_of(lax.rem(base, S), TPW)

    pltpu.sync_copy(ids_hbm.at[pl.ds(base, TPW)], idx_v)
    pltpu.sync_copy(tt_hbm.at[pl.ds(0, 1)], tt_v)
    pltpu.sync_copy(gamma_hbm, gamma_v)
    pltpu.sync_copy(beta_hbm, beta_v)

    inv_h = jnp.float32(1.0 / H)

    for c in range(NCHUNK):
        pltpu.async_copy(word_hbm.at[idx_v.at[pl.ds(c * CH, CH)]], rows_v,
                         sem).wait()
        pltpu.sync_copy(pos_hbm.at[pl.ds(s0 + c * CH, CH)], pos_v)

        def row_fn(r, _):
            def stat_fn(j, carry):
                acc, accsq = carry
                off = pl.multiple_of(j * LANES, LANES)
                x = (rows_v[r, pl.ds(off, LANES)]
                     + pos_v[r, pl.ds(off, LANES)]
                     + tt_v[0, pl.ds(off, LANES)])
                rows_v[r, pl.ds(off, LANES)] = x
                return acc + x, accsq + x * x

            zero = jnp.zeros((LANES,), jnp.float32)
            acc, accsq = lax.fori_loop(0, HCHUNKS, stat_fn, (zero, zero))
            mean = jnp.sum(acc) * inv_h
            var = jnp.sum(accsq) * inv_h - mean * mean
            rstd = _rsqrt(var + jnp.float32(1e-5))

            def norm_fn(j, _):
                off = pl.multiple_of(j * LANES, LANES)
                x = rows_v[r, pl.ds(off, LANES)]
                g = gamma_v[pl.ds(off, LANES)]
                bta = beta_v[pl.ds(off, LANES)]
                rows_v[r, pl.ds(off, LANES)] = (x - mean) * rstd * g + bta
                return 0

            lax.fori_loop(0, HCHUNKS, norm_fn, 0)
            return 0

        lax.fori_loop(0, CH, row_fn, 0)
        pltpu.sync_copy(rows_v, out_hbm.at[pl.ds(base + c * CH, CH)])


@jax.jit
def _run(ids_flat, word, pos, tt, gamma, beta):
    mesh = plsc.VectorSubcoreMesh(core_axis_name="c", subcore_axis_name="s")
    k = functools.partial(
        pl.kernel,
        out_type=jax.ShapeDtypeStruct((NT, H), jnp.float32),
        mesh=mesh,
        scratch_types=[
            pltpu.VMEM((TPW,), jnp.int32),
            pltpu.VMEM((CH, H), jnp.float32),
            pltpu.VMEM((CH, H), jnp.float32),
            pltpu.VMEM((1, H), jnp.float32),
            pltpu.VMEM((H,), jnp.float32),
            pltpu.VMEM((H,), jnp.float32),
            pltpu.SemaphoreType.DMA,
        ],
        compiler_params=pltpu.CompilerParams(needs_layout_passes=False),
    )(_body)
    return k(ids_flat, word, pos, tt, gamma, beta)


def kernel(input_ids, word_embeddings, position_embeddings,
           token_type_embeddings, ln_gamma, ln_beta):
    ids_flat = input_ids.reshape(NT).astype(jnp.int32)
    out = _run(ids_flat, word_embeddings, position_embeddings,
               token_type_embeddings, ln_gamma, ln_beta)
    return out.reshape(B, S, H)


# speedup vs baseline: 1.2554x; 1.2554x over previous
import functools

import jax
import jax.numpy as jnp
from jax import lax
from jax.experimental import pallas as pl
from jax.experimental.pallas import tpu as pltpu
from jax.experimental.pallas import tpu_sc as plsc

V = 100000
P = 8192
H = 1024
B = 4
S = 2048

NC = 2
NS = 16
NW = NC * NS
NT = B * S
TPW = NT // NW
CH = 16
NCHUNK = TPW // CH
LANES = 16
HCHUNKS = H // LANES
NACC = 4


def _rsqrt(x):
    i = lax.bitcast_convert_type(x, jnp.int32)
    i = jnp.int32(0x5F3759DF) - lax.shift_right_arithmetic(i, jnp.int32(1))
    y = lax.bitcast_convert_type(i, jnp.float32)
    for _ in range(3):
        y = y * (jnp.float32(1.5) - jnp.float32(0.5) * x * y * y)
    return y


def _body(ids_hbm, word_hbm, pos_hbm, tt_hbm, gamma_hbm, beta_hbm, out_hbm,
          idx_v, rows_a, rows_b, pos_a, pos_b, tt_v, gamma_v, beta_v,
          gsem_a, gsem_b, psem_a, psem_b, osem_a, osem_b):
    cid = lax.axis_index("c")
    sid = lax.axis_index("s")
    wid = sid * NC + cid
    base = pl.multiple_of(wid * TPW, TPW)
    s0 = pl.multiple_of(lax.rem(base, S), TPW)

    pltpu.sync_copy(ids_hbm.at[pl.ds(base, TPW)], idx_v)
    pltpu.sync_copy(tt_hbm.at[pl.ds(0, 1)], tt_v)
    pltpu.sync_copy(gamma_hbm, gamma_v)
    pltpu.sync_copy(beta_hbm, beta_v)

    inv_h = jnp.float32(1.0 / H)

    def compute(rows_v, pos_v):
        def row_fn(r, _):
            acc = [jnp.zeros((LANES,), jnp.float32) for _ in range(NACC)]
            accsq = [jnp.zeros((LANES,), jnp.float32) for _ in range(NACC)]
            for j in range(HCHUNKS):
                sl = pl.ds(j * LANES, LANES)
                x = rows_v[r, sl] + pos_v[r, sl] + tt_v[0, sl]
                rows_v[r, sl] = x
                acc[j % NACC] = acc[j % NACC] + x
                accsq[j % NACC] = accsq[j % NACC] + x * x
            asum = (acc[0] + acc[1]) + (acc[2] + acc[3])
            sqsum = (accsq[0] + accsq[1]) + (accsq[2] + accsq[3])
            mean = jnp.sum(asum) * inv_h
            var = jnp.sum(sqsum) * inv_h - mean * mean
            rstd = _rsqrt(var + jnp.float32(1e-5))
            mrs = mean * rstd
            for j in range(HCHUNKS):
                sl = pl.ds(j * LANES, LANES)
                x = rows_v[r, sl]
                rows_v[r, sl] = (x * rstd - mrs) * gamma_v[sl] + beta_v[sl]
            return 0

        lax.fori_loop(0, CH, row_fn, 0)

    def super_fn(i, _):
        ca = pl.multiple_of(2 * i * CH, CH)
        cb = pl.multiple_of((2 * i + 1) * CH, CH)

        @pl.when(i > 0)
        def _drain():
            pltpu.make_async_copy(
                rows_a, out_hbm.at[pl.ds(base, CH)], osem_a).wait()
            pltpu.make_async_copy(
                rows_b, out_hbm.at[pl.ds(base, CH)], osem_b).wait()

        ga = pltpu.async_copy(word_hbm.at[idx_v.at[pl.ds(ca, CH)]],
                              rows_a, gsem_a)
        pa = pltpu.async_copy(pos_hbm.at[pl.ds(s0 + ca, CH)], pos_a, psem_a)
        gb = pltpu.async_copy(word_hbm.at[idx_v.at[pl.ds(cb, CH)]],
                              rows_b, gsem_b)
        pb = pltpu.async_copy(pos_hbm.at[pl.ds(s0 + cb, CH)], pos_b, psem_b)

        ga.wait()
        pa.wait()
        compute(rows_a, pos_a)
        pltpu.async_copy(rows_a, out_hbm.at[pl.ds(base + ca, CH)], osem_a)

        gb.wait()
        pb.wait()
        compute(rows_b, pos_b)
        pltpu.async_copy(rows_b, out_hbm.at[pl.ds(base + cb, CH)], osem_b)
        return 0

    lax.fori_loop(0, NCHUNK // 2, super_fn, 0)
    pltpu.make_async_copy(rows_a, out_hbm.at[pl.ds(base, CH)], osem_a).wait()
    pltpu.make_async_copy(rows_b, out_hbm.at[pl.ds(base, CH)], osem_b).wait()


@jax.jit
def _run(ids_flat, word, pos, tt, gamma, beta):
    mesh = plsc.VectorSubcoreMesh(core_axis_name="c", subcore_axis_name="s")
    k = functools.partial(
        pl.kernel,
        out_type=jax.ShapeDtypeStruct((NT, H), jnp.float32),
        mesh=mesh,
        scratch_types=[
            pltpu.VMEM((TPW,), jnp.int32),
            pltpu.VMEM((CH, H), jnp.float32),
            pltpu.VMEM((CH, H), jnp.float32),
            pltpu.VMEM((CH, H), jnp.float32),
            pltpu.VMEM((CH, H), jnp.float32),
            pltpu.VMEM((1, H), jnp.float32),
            pltpu.VMEM((H,), jnp.float32),
            pltpu.VMEM((H,), jnp.float32),
            pltpu.SemaphoreType.DMA,
            pltpu.SemaphoreType.DMA,
            pltpu.SemaphoreType.DMA,
            pltpu.SemaphoreType.DMA,
            pltpu.SemaphoreType.DMA,
            pltpu.SemaphoreType.DMA,
        ],
        compiler_params=pltpu.CompilerParams(needs_layout_passes=False),
    )(_body)
    return k(ids_flat, word, pos, tt, gamma, beta)


def kernel(input_ids, word_embeddings, position_embeddings,
           token_type_embeddings, ln_gamma, ln_beta):
    ids_flat = input_ids.reshape(NT).astype(jnp.int32)
    out = _run(ids_flat, word_embeddings, position_embeddings,
               token_type_embeddings, ln_gamma, ln_beta)
    return out.reshape(B, S, H)


# speedup vs baseline: 2.9170x; 2.3236x over previous
import functools

import jax
import jax.numpy as jnp
from jax import lax
from jax.experimental import pallas as pl
from jax.experimental.pallas import tpu as pltpu
from jax.experimental.pallas import tpu_sc as plsc

V = 100000
P = 8192
H = 1024
B = 4
S = 2048

NC = 2
NS = 16
NW = NC * NS
NT = B * S
TPW = NT // NW
CH = 32
NCHUNK = TPW // CH

TOK_BLK = 128
POS_BLKS = S // TOK_BLK


def _gather_body(ids_hbm, word_hbm, out_hbm,
                 idx_v, rows_a, rows_b, gsem_a, gsem_b, osem_a, osem_b):
    cid = lax.axis_index("c")
    sid = lax.axis_index("s")
    wid = sid * NC + cid
    base = pl.multiple_of(wid * TPW, TPW)

    pltpu.sync_copy(ids_hbm.at[pl.ds(base, TPW)], idx_v)

    def super_fn(i, _):
        ca = pl.multiple_of(2 * i * CH, CH)
        cb = pl.multiple_of((2 * i + 1) * CH, CH)

        @pl.when(i > 0)
        def _drain():
            pltpu.make_async_copy(
                rows_a, out_hbm.at[pl.ds(base, CH)], osem_a).wait()
            pltpu.make_async_copy(
                rows_b, out_hbm.at[pl.ds(base, CH)], osem_b).wait()

        ga = pltpu.async_copy(word_hbm.at[idx_v.at[pl.ds(ca, CH)]],
                              rows_a, gsem_a)
        gb = pltpu.async_copy(word_hbm.at[idx_v.at[pl.ds(cb, CH)]],
                              rows_b, gsem_b)
        ga.wait()
        pltpu.async_copy(rows_a, out_hbm.at[pl.ds(base + ca, CH)], osem_a)
        gb.wait()
        pltpu.async_copy(rows_b, out_hbm.at[pl.ds(base + cb, CH)], osem_b)
        return 0

    lax.fori_loop(0, NCHUNK // 2, super_fn, 0)
    pltpu.make_async_copy(rows_a, out_hbm.at[pl.ds(base, CH)], osem_a).wait()
    pltpu.make_async_copy(rows_b, out_hbm.at[pl.ds(base, CH)], osem_b).wait()


def _ln_body(g_ref, p_ref, tt_ref, gamma_ref, beta_ref, o_ref):
    x = g_ref[...] + p_ref[...] + tt_ref[...]
    mean = jnp.mean(x, axis=-1, keepdims=True)
    d = x - mean
    var = jnp.mean(d * d, axis=-1, keepdims=True)
    y = d * lax.rsqrt(var + jnp.float32(1e-5))
    o_ref[...] = y * gamma_ref[...] + beta_ref[...]


@jax.jit
def _run(ids_flat, word, pos, tt, gamma, beta):
    mesh = plsc.VectorSubcoreMesh(core_axis_name="c", subcore_axis_name="s")
    gathered = functools.partial(
        pl.kernel,
        out_type=jax.ShapeDtypeStruct((NT, H), jnp.float32),
        mesh=mesh,
        scratch_types=[
            pltpu.VMEM((TPW,), jnp.int32),
            pltpu.VMEM((CH, H), jnp.float32),
            pltpu.VMEM((CH, H), jnp.float32),
            pltpu.SemaphoreType.DMA,
            pltpu.SemaphoreType.DMA,
            pltpu.SemaphoreType.DMA,
            pltpu.SemaphoreType.DMA,
        ],
        compiler_params=pltpu.CompilerParams(needs_layout_passes=False),
    )(_gather_body)(ids_flat, word)

    tt_row = tt[0:1, :]
    gamma2 = gamma.reshape(1, H)
    beta2 = beta.reshape(1, H)
    out = pl.pallas_call(
        _ln_body,
        grid=(NT // TOK_BLK,),
        in_specs=[
            pl.BlockSpec((TOK_BLK, H), lambda t: (t, 0)),
            pl.BlockSpec((TOK_BLK, H), lambda t: (t % POS_BLKS, 0)),
            pl.BlockSpec((1, H), lambda t: (0, 0)),
            pl.BlockSpec((1, H), lambda t: (0, 0)),
            pl.BlockSpec((1, H), lambda t: (0, 0)),
        ],
        out_specs=pl.BlockSpec((TOK_BLK, H), lambda t: (t, 0)),
        out_shape=jax.ShapeDtypeStruct((NT, H), jnp.float32),
    )(gathered, pos[:S], tt_row, gamma2, beta2)
    return out


def kernel(input_ids, word_embeddings, position_embeddings,
           token_type_embeddings, ln_gamma, ln_beta):
    ids_flat = input_ids.reshape(NT).astype(jnp.int32)
    out = _run(ids_flat, word_embeddings, position_embeddings,
               token_type_embeddings, ln_gamma, ln_beta)
    return out.reshape(B, S, H)


# speedup vs baseline: 3.4865x; 1.1952x over previous
import functools

import jax
import jax.numpy as jnp
from jax import lax
from jax.experimental import pallas as pl
from jax.experimental.pallas import tpu as pltpu
from jax.experimental.pallas import tpu_sc as plsc

V = 100000
P = 8192
H = 1024
B = 4
S = 2048

NC = 2
NS = 16
NW = NC * NS
NT = B * S
TPW = NT // NW
CH = 32
NCHUNK = TPW // CH

TOK_BLK = 256
POS_BLKS = S // TOK_BLK


def _gather_body(ids_hbm, word_hbm, out_hbm,
                 idx_v, rows_a, rows_b, gsem_a, gsem_b, osem_a, osem_b):
    cid = lax.axis_index("c")
    sid = lax.axis_index("s")
    wid = sid * NC + cid
    base = pl.multiple_of(wid * TPW, TPW)

    pltpu.sync_copy(ids_hbm.at[pl.ds(base, TPW)], idx_v)

    def super_fn(i, _):
        ca = pl.multiple_of(2 * i * CH, CH)
        cb = pl.multiple_of((2 * i + 1) * CH, CH)

        @pl.when(i > 0)
        def _drain():
            pltpu.make_async_copy(
                rows_a, out_hbm.at[pl.ds(base, CH)], osem_a).wait()
            pltpu.make_async_copy(
                rows_b, out_hbm.at[pl.ds(base, CH)], osem_b).wait()

        ga = pltpu.async_copy(word_hbm.at[idx_v.at[pl.ds(ca, CH)]],
                              rows_a, gsem_a)
        gb = pltpu.async_copy(word_hbm.at[idx_v.at[pl.ds(cb, CH)]],
                              rows_b, gsem_b)
        ga.wait()
        pltpu.async_copy(rows_a, out_hbm.at[pl.ds(base + ca, CH)], osem_a)
        gb.wait()
        pltpu.async_copy(rows_b, out_hbm.at[pl.ds(base + cb, CH)], osem_b)
        return 0

    lax.fori_loop(0, NCHUNK // 2, super_fn, 0)
    pltpu.make_async_copy(rows_a, out_hbm.at[pl.ds(base, CH)], osem_a).wait()
    pltpu.make_async_copy(rows_b, out_hbm.at[pl.ds(base, CH)], osem_b).wait()


def _ln_body(g_ref, p_ref, tt_ref, gamma_ref, beta_ref, o_ref):
    x = g_ref[...] + p_ref[...] + tt_ref[...]
    mean = jnp.mean(x, axis=-1, keepdims=True)
    d = x - mean
    var = jnp.mean(d * d, axis=-1, keepdims=True)
    y = d * lax.rsqrt(var + jnp.float32(1e-5))
    o_ref[...] = y * gamma_ref[...] + beta_ref[...]


@jax.jit
def _run(ids_flat, word, pos, tt, gamma, beta):
    mesh = plsc.VectorSubcoreMesh(core_axis_name="c", subcore_axis_name="s")
    gathered = functools.partial(
        pl.kernel,
        out_type=jax.ShapeDtypeStruct((NT, H), jnp.float32),
        mesh=mesh,
        scratch_types=[
            pltpu.VMEM((TPW,), jnp.int32),
            pltpu.VMEM((CH, H), jnp.float32),
            pltpu.VMEM((CH, H), jnp.float32),
            pltpu.SemaphoreType.DMA,
            pltpu.SemaphoreType.DMA,
            pltpu.SemaphoreType.DMA,
            pltpu.SemaphoreType.DMA,
        ],
        compiler_params=pltpu.CompilerParams(needs_layout_passes=False),
    )(_gather_body)(ids_flat, word)

    tt_row = tt[0:1, :]
    gamma2 = gamma.reshape(1, H)
    beta2 = beta.reshape(1, H)
    out = pl.pallas_call(
        _ln_body,
        grid=(POS_BLKS, B),
        in_specs=[
            pl.BlockSpec((TOK_BLK, H), lambda p, b: (b * POS_BLKS + p, 0)),
            pl.BlockSpec((TOK_BLK, H), lambda p, b: (p, 0)),
            pl.BlockSpec((1, H), lambda p, b: (0, 0)),
            pl.BlockSpec((1, H), lambda p, b: (0, 0)),
            pl.BlockSpec((1, H), lambda p, b: (0, 0)),
        ],
        out_specs=pl.BlockSpec((TOK_BLK, H), lambda p, b: (b * POS_BLKS + p, 0)),
        out_shape=jax.ShapeDtypeStruct((NT, H), jnp.float32),
    )(gathered, pos[:S], tt_row, gamma2, beta2)
    return out


def kernel(input_ids, word_embeddings, position_embeddings,
           token_type_embeddings, ln_gamma, ln_beta):
    ids_flat = input_ids.reshape(NT).astype(jnp.int32)
    out = _run(ids_flat, word_embeddings, position_embeddings,
               token_type_embeddings, ln_gamma, ln_beta)
    return out.reshape(B, S, H)


# speedup vs baseline: 3.8949x; 1.1172x over previous
import functools

import jax
import jax.numpy as jnp
from jax import lax
from jax.experimental import pallas as pl
from jax.experimental.pallas import tpu as pltpu
from jax.experimental.pallas import tpu_sc as plsc

V = 100000
P = 8192
H = 1024
B = 4
S = 2048

NC = 2
NS = 16
NW = NC * NS
NT = B * S
TPW = NT // NW
CH = 32
NCHUNK = TPW // CH

TOK_BLK = 512
POS_BLKS = S // TOK_BLK


def _gather_body(ids_hbm, word_hbm, out_hbm,
                 idx_v, rows_a, rows_b, gsem_a, gsem_b, osem_a, osem_b):
    cid = lax.axis_index("c")
    sid = lax.axis_index("s")
    wid = sid * NC + cid
    base = pl.multiple_of(wid * TPW, TPW)

    pltpu.sync_copy(ids_hbm.at[pl.ds(base, TPW)], idx_v)

    def super_fn(i, _):
        ca = pl.multiple_of(2 * i * CH, CH)
        cb = pl.multiple_of((2 * i + 1) * CH, CH)

        @pl.when(i > 0)
        def _drain():
            pltpu.make_async_copy(
                rows_a, out_hbm.at[pl.ds(base, CH)], osem_a).wait()
            pltpu.make_async_copy(
                rows_b, out_hbm.at[pl.ds(base, CH)], osem_b).wait()

        ga = pltpu.async_copy(word_hbm.at[idx_v.at[pl.ds(ca, CH)]],
                              rows_a, gsem_a)
        gb = pltpu.async_copy(word_hbm.at[idx_v.at[pl.ds(cb, CH)]],
                              rows_b, gsem_b)
        ga.wait()
        pltpu.async_copy(rows_a, out_hbm.at[pl.ds(base + ca, CH)], osem_a)
        gb.wait()
        pltpu.async_copy(rows_b, out_hbm.at[pl.ds(base + cb, CH)], osem_b)
        return 0

    lax.fori_loop(0, NCHUNK // 2, super_fn, 0)
    pltpu.make_async_copy(rows_a, out_hbm.at[pl.ds(base, CH)], osem_a).wait()
    pltpu.make_async_copy(rows_b, out_hbm.at[pl.ds(base, CH)], osem_b).wait()


def _ln_body(g_ref, p_ref, tt_ref, gamma_ref, beta_ref, o_ref):
    x = g_ref[...] + p_ref[...] + tt_ref[...]
    mean = jnp.mean(x, axis=-1, keepdims=True)
    d = x - mean
    var = jnp.mean(d * d, axis=-1, keepdims=True)
    y = d * lax.rsqrt(var + jnp.float32(1e-5))
    o_ref[...] = y * gamma_ref[...] + beta_ref[...]


@jax.jit
def _run(ids_flat, word, pos, tt, gamma, beta):
    mesh = plsc.VectorSubcoreMesh(core_axis_name="c", subcore_axis_name="s")
    gathered = functools.partial(
        pl.kernel,
        out_type=jax.ShapeDtypeStruct((NT, H), jnp.float32),
        mesh=mesh,
        scratch_types=[
            pltpu.VMEM((TPW,), jnp.int32),
            pltpu.VMEM((CH, H), jnp.float32),
            pltpu.VMEM((CH, H), jnp.float32),
            pltpu.SemaphoreType.DMA,
            pltpu.SemaphoreType.DMA,
            pltpu.SemaphoreType.DMA,
            pltpu.SemaphoreType.DMA,
        ],
        compiler_params=pltpu.CompilerParams(needs_layout_passes=False),
    )(_gather_body)(ids_flat, word)

    tt_row = tt[0:1, :]
    gamma2 = gamma.reshape(1, H)
    beta2 = beta.reshape(1, H)
    out = pl.pallas_call(
        _ln_body,
        grid=(POS_BLKS, B),
        in_specs=[
            pl.BlockSpec((TOK_BLK, H), lambda p, b: (b * POS_BLKS + p, 0)),
            pl.BlockSpec((TOK_BLK, H), lambda p, b: (p, 0)),
            pl.BlockSpec((1, H), lambda p, b: (0, 0)),
            pl.BlockSpec((1, H), lambda p, b: (0, 0)),
            pl.BlockSpec((1, H), lambda p, b: (0, 0)),
        ],
        out_specs=pl.BlockSpec((TOK_BLK, H), lambda p, b: (b * POS_BLKS + p, 0)),
        out_shape=jax.ShapeDtypeStruct((NT, H), jnp.float32),
    )(gathered, pos[:S], tt_row, gamma2, beta2)
    return out


def kernel(input_ids, word_embeddings, position_embeddings,
           token_type_embeddings, ln_gamma, ln_beta):
    ids_flat = input_ids.reshape(NT).astype(jnp.int32)
    out = _run(ids_flat, word_embeddings, position_embeddings,
               token_type_embeddings, ln_gamma, ln_beta)
    return out.reshape(B, S, H)
